# Initial kernel scaffold; baseline (speedup 1.0000x reference)
import jax
import jax.numpy as jnp
from jax.experimental import pallas as pl


def kernel(x, table):
    raise NotImplementedError("write your pallas kernel here")


# speedup vs baseline: 4.5022x; 4.5022x over previous
import jax
import jax.numpy as jnp
from jax import lax
from jax.experimental import pallas as pl
from jax.experimental.pallas import tpu as pltpu
from jax.experimental.pallas import tpu_sc as plsc

BATCH = 4096
HIST = 50
EMB_DIM = 64

NC = 2
NS = 16
NW = NC * NS

TOTAL = BATCH * HIST
PER_W = TOTAL // NW
SUB = 128
CHUNK = 640
NSUB = CHUNK // SUB
NCHUNK = PER_W // CHUNK


def _body(idx_hbm, table_hbm, out_hbm, idx_v, rows_v, sem):
    wid = lax.axis_index("s") * NC + lax.axis_index("c")

    def chunk(g, carry):
        base = wid * PER_W + g * CHUNK
        pltpu.sync_copy(idx_hbm.at[pl.ds(base, CHUNK)], idx_v)
        copies = [
            pltpu.async_copy(
                table_hbm.at[idx_v.at[pl.ds(j * SUB, SUB)]],
                rows_v.at[pl.ds(j * SUB, SUB)],
                sem,
            )
            for j in range(NSUB)
        ]
        for c in copies:
            c.wait()
        pltpu.sync_copy(rows_v, out_hbm.at[pl.ds(base, CHUNK)])
        return carry

    lax.fori_loop(0, NCHUNK, chunk, 0)


@jax.jit
def _gather(idx, table):
    run = pl.kernel(
        _body,
        out_type=jax.ShapeDtypeStruct((TOTAL, EMB_DIM), jnp.float32),
        mesh=plsc.VectorSubcoreMesh(core_axis_name="c", subcore_axis_name="s"),
        compiler_params=pltpu.CompilerParams(use_tc_tiling_on_sc=False),
        scratch_types=[
            pltpu.VMEM((CHUNK,), jnp.int32),
            pltpu.VMEM((CHUNK, EMB_DIM), jnp.float32),
            pltpu.SemaphoreType.DMA,
        ],
    )
    return run(idx, table)


def kernel(x, table):
    idx = x.astype(jnp.int32).reshape(TOTAL)
    out = _gather(idx, table)
    return out.reshape(BATCH, HIST, EMB_DIM)


# speedup vs baseline: 4.6199x; 1.0261x over previous
import jax
import jax.numpy as jnp
from jax import lax
from jax.experimental import pallas as pl
from jax.experimental.pallas import tpu as pltpu
from jax.experimental.pallas import tpu_sc as plsc

BATCH = 4096
HIST = 50
EMB_DIM = 64

NC = 2
NS = 16
NW = NC * NS

TOTAL = BATCH * HIST
PER_W = TOTAL // NW
SUB = 128
CHUNK = 640
NSUB = CHUNK // SUB
NCHUNK = PER_W // CHUNK
NPAIR = NCHUNK // 2


def _body(idx_hbm, table_hbm, out_hbm,
          idx_v, rows0, rows1, sem_g, sem_o0, sem_o1):
    wid = lax.axis_index("s") * NC + lax.axis_index("c")
    base_w = wid * PER_W

    pltpu.sync_copy(idx_hbm.at[pl.ds(base_w, PER_W)], idx_v)

    def gather_chunk(g, rows):
        copies = [
            pltpu.async_copy(
                table_hbm.at[idx_v.at[pl.ds(g * CHUNK + j * SUB, SUB)]],
                rows.at[pl.ds(j * SUB, SUB)],
                sem_g,
            )
            for j in range(NSUB)
        ]
        for c in copies:
            c.wait()

    def store_chunk(g, rows, sem):
        pltpu.make_async_copy(
            rows, out_hbm.at[pl.ds(base_w + g * CHUNK, CHUNK)], sem
        ).start()

    def wait_store(g, rows, sem):
        pltpu.make_async_copy(
            rows, out_hbm.at[pl.ds(base_w + g * CHUNK, CHUNK)], sem
        ).wait()

    def pair(h, carry):
        g0 = 2 * h
        g1 = 2 * h + 1

        @pl.when(h > 0)
        def _():
            wait_store(g0, rows0, sem_o0)

        gather_chunk(g0, rows0)

        @pl.when(h > 0)
        def _():
            wait_store(g1, rows1, sem_o1)

        store_chunk(g0, rows0, sem_o0)
        gather_chunk(g1, rows1)
        store_chunk(g1, rows1, sem_o1)
        return carry

    lax.fori_loop(0, NPAIR, pair, 0)
    wait_store(NCHUNK - 2, rows0, sem_o0)
    wait_store(NCHUNK - 1, rows1, sem_o1)


@jax.jit
def _gather(idx, table):
    run = pl.kernel(
        _body,
        out_type=jax.ShapeDtypeStruct((TOTAL, EMB_DIM), jnp.float32),
        mesh=plsc.VectorSubcoreMesh(core_axis_name="c", subcore_axis_name="s"),
        compiler_params=pltpu.CompilerParams(use_tc_tiling_on_sc=False),
        scratch_types=[
            pltpu.VMEM((PER_W,), jnp.int32),
            pltpu.VMEM((CHUNK, EMB_DIM), jnp.float32),
            pltpu.VMEM((CHUNK, EMB_DIM), jnp.float32),
            pltpu.SemaphoreType.DMA,
            pltpu.SemaphoreType.DMA,
            pltpu.SemaphoreType.DMA,
        ],
    )
    return run(idx, table)


def kernel(x, table):
    idx = x.astype(jnp.int32).reshape(TOTAL)
    out = _gather(idx, table)
    return out.reshape(BATCH, HIST, EMB_DIM)
